# baseline (speedup 1.0000x reference)
import functools

import jax
import jax.numpy as jnp
from jax import lax
from jax.experimental import pallas as pl
from jax.experimental.pallas import tpu as pltpu, tpu_sc as plsc

N = 10000
E = 160000
P = 100000
NUM_TRAIN = 80000

E_PAD = 163840
P_PAD = 102400
BLK = 80
BM = 400
GRID_M = N // BM

def _mesh():
    return plsc.VectorSubcoreMesh(core_axis_name="c", subcore_axis_name="s")


def _make_agg(C):
    split_edges = (C == 1)
    if split_edges:
        edges_per_worker = E_PAD // 32
        out_rows = 2 * N
    else:
        edges_per_worker = E_PAD // 16
        out_rows = C * N
    nb = edges_per_worker // BLK
    chunks_per_core = 1 if split_edges else C // 2
    zrows = 16
    nrowblocks = N // 80
    ER = 6

    @functools.partial(
        pl.kernel,
        out_type=jax.ShapeDtypeStruct((out_rows, 128), jnp.float32),
        mesh=_mesh(),
        scratch_types=[
            pltpu.VMEM((ER, BLK), jnp.int32),
            pltpu.VMEM((ER, BLK), jnp.int32),
            pltpu.VMEM((ER, BLK), jnp.float32),
            pltpu.VMEM((4, BLK), jnp.int32),
            pltpu.VMEM((4, BLK, 128), jnp.float32),
            pltpu.VMEM((zrows, 128), jnp.float32),
            pltpu.VMEM_SHARED((N, 128), jnp.float32),
            pltpu.SemaphoreType.DMA((ER,)),
            pltpu.SemaphoreType.DMA((ER,)),
            pltpu.SemaphoreType.DMA((ER,)),
            pltpu.SemaphoreType.DMA((4,)),
            pltpu.SemaphoreType.DMA((4,)),
        ],
    )
    def agg(src_hbm, dst_hbm, w_hbm, table_hbm, out_hbm,
            src_r, dst_r, w_r, idx_v, rows_v, zero_v, acc_sh,
            esrc, edst, ew_sem, gsem, ssem):
        c = lax.axis_index("c")
        s = lax.axis_index("s")
        if split_edges:
            ebase = (c * 16 + s) * edges_per_worker
        else:
            ebase = s * edges_per_worker
        ebase = pl.multiple_of(ebase, 128)
        for j in range(zrows):
            for k in range(8):
                zero_v[j, pl.ds(k * 16, 16)] = jnp.zeros((16,), jnp.float32)

        def estage(b):
            pe = lax.rem(b, ER)
            off = pl.multiple_of(ebase + b * BLK, 8)
            pltpu.async_copy(src_hbm.at[pl.ds(off, BLK)], src_r.at[pe],
                             esrc.at[pe])
            pltpu.async_copy(dst_hbm.at[pl.ds(off, BLK)], dst_r.at[pe],
                             edst.at[pe])
            pltpu.async_copy(w_hbm.at[pl.ds(off, BLK)], w_r.at[pe],
                             ew_sem.at[pe])

        def ewait(b):
            pe = lax.rem(b, ER)
            off = pl.multiple_of(ebase + b * BLK, 8)
            pltpu.make_async_copy(src_hbm.at[pl.ds(off, BLK)], src_r.at[pe],
                                  esrc.at[pe]).wait()
            pltpu.make_async_copy(dst_hbm.at[pl.ds(off, BLK)], dst_r.at[pe],
                                  edst.at[pe]).wait()
            pltpu.make_async_copy(w_hbm.at[pl.ds(off, BLK)], w_r.at[pe],
                                  ew_sem.at[pe]).wait()

        trips = (nrowblocks - 1 - s) // 16 + 1

        def run_chunk(chunk, out_base):
            def zbody(i, carry):
                off = pl.multiple_of((s + i * 16) * 80, 8)
                for rr in range(80 // zrows):
                    pltpu.sync_copy(zero_v,
                                    acc_sh.at[pl.ds(off + rr * zrows, zrows)])
                return carry
            lax.fori_loop(0, trips, zbody, 0)
            plsc.subcore_barrier()
            cbase = chunk * N

            def fill_idx(b, p):
                pe = lax.rem(b, ER)
                for k in range(BLK // 16):
                    idx_v[p, pl.ds(k * 16, 16)] = (
                        src_r[pe, pl.ds(k * 16, 16)] + cbase)

            def start_gather(b, p):
                fill_idx(b, p)
                pltpu.async_copy(table_hbm.at[idx_v.at[p]], rows_v.at[p],
                                 gsem.at[p])

            def scale(b, p):
                pe = lax.rem(b, ER)

                @plsc.parallel_loop(0, BLK // 16, unroll=2)
                def _(g):
                    w16 = w_r[pe, pl.ds(g * 16, 16)]
                    for i in range(16):
                        ws = w16.at[jnp.full((16,), i, jnp.int32)].get(
                            mode="promise_in_bounds")
                        vals = [rows_v[p, g * 16 + i, pl.ds(k * 16, 16)] * ws
                                for k in range(8)]
                        for k in range(8):
                            rows_v[p, g * 16 + i, pl.ds(k * 16, 16)] = vals[k]

            def start_scatter(b, p):
                pe = lax.rem(b, ER)
                for g in range(BLK // 16):
                    dvec = dst_r[pe, pl.ds(g * 16, 16)]
                    pltpu.async_copy(rows_v.at[p, pl.ds(g * 16, 16)],
                                     acc_sh.at[dvec], ssem.at[p], add=True)

            def wait_scatter(p):
                dvec = dst_r[0, pl.ds(0, 16)]
                for g in range(BLK // 16):
                    pltpu.make_async_copy(rows_v.at[p, pl.ds(g * 16, 16)],
                                          acc_sh.at[dvec],
                                          ssem.at[p]).wait()

            for b0 in range(5):
                estage(b0)
            for b0 in range(2):
                ewait(b0)
                start_gather(b0, b0)

            def body(b, carry):
                p = lax.rem(b, 4)

                @pl.when(b + 5 < nb)
                def _():
                    estage(b + 5)
                pltpu.make_async_copy(table_hbm.at[idx_v.at[p]],
                                      rows_v.at[p], gsem.at[p]).wait()

                @pl.when(b + 2 < nb)
                def _():
                    q = lax.rem(b + 2, 4)
                    ewait(b + 2)

                    @pl.when(b >= 2)
                    def _():
                        wait_scatter(q)
                    start_gather(b + 2, q)
                start_scatter(b, p)
                return carry
            lax.fori_loop(0, nb, body, 0)
            for t in (nb - 4, nb - 3, nb - 2, nb - 1):
                wait_scatter(t % 4)
            plsc.subcore_barrier()

            def obody(i, carry):
                off = pl.multiple_of((s + i * 16) * 80, 8)
                pltpu.sync_copy(
                    acc_sh.at[pl.ds(off, 80)],
                    out_hbm.at[pl.ds(pl.multiple_of(out_base + off, 8), 80)])
                return carry
            lax.fori_loop(0, trips, obody, 0)
            plsc.subcore_barrier()

        if split_edges:
            run_chunk(0, c * N)
        else:
            def chunk_body(kk, carry):
                chunk = kk * 2 + c
                run_chunk(chunk, chunk * N)
                return carry
            lax.fori_loop(0, chunks_per_core, chunk_body, 0)

    return agg


_agg_cache = {}


def _agg(C):
    if C not in _agg_cache:
        _agg_cache[C] = _make_agg(C)
    return _agg_cache[C]


def _make_pair_mean():
  @functools.partial(
    pl.kernel,
    out_type=jax.ShapeDtypeStruct((P_PAD, 128), jnp.float32),
    mesh=_mesh(),
    scratch_types=[
        pltpu.VMEM((P_PAD // 32,), jnp.int32),
        pltpu.VMEM((P_PAD // 32,), jnp.int32),
        pltpu.VMEM((2, BLK, 128), jnp.float32),
        pltpu.VMEM((2, BLK, 128), jnp.float32),
        pltpu.SemaphoreType.DMA((2,)),
        pltpu.SemaphoreType.DMA((2,)),
        pltpu.SemaphoreType.DMA((2,)),
    ],
  )
  def _pair_mean(l_hbm, r_hbm, emb_hbm, out_hbm,
               l_v, r_v, lrows, rrows, lsem, rsem, osem):
    c = lax.axis_index("c")
    s = lax.axis_index("s")
    wid = c * 16 + s
    per_w = P_PAD // 32
    nb = per_w // BLK
    base = pl.multiple_of(wid * per_w, 128)
    pltpu.sync_copy(l_hbm.at[pl.ds(base, per_w)], l_v)
    pltpu.sync_copy(r_hbm.at[pl.ds(base, per_w)], r_v)

    def start(b, p):
        pltpu.async_copy(emb_hbm.at[l_v.at[pl.ds(b * BLK, BLK)]],
                         lrows.at[p], lsem.at[p])
        pltpu.async_copy(emb_hbm.at[r_v.at[pl.ds(b * BLK, BLK)]],
                         rrows.at[p], rsem.at[p])

    start(0, 0)
    start(1, 1)

    def body(b, carry):
        p = lax.rem(b, 2)
        pltpu.make_async_copy(emb_hbm.at[l_v.at[pl.ds(b * BLK, BLK)]],
                              lrows.at[p], lsem.at[p]).wait()
        pltpu.make_async_copy(emb_hbm.at[r_v.at[pl.ds(b * BLK, BLK)]],
                              rrows.at[p], rsem.at[p]).wait()

        def combine(e, carry2):
            for k in range(8):
                lrows[p, e, pl.ds(k * 16, 16)] = (
                    lrows[p, e, pl.ds(k * 16, 16)]
                    + rrows[p, e, pl.ds(k * 16, 16)]) * 0.5
            return carry2
        lax.fori_loop(0, BLK, combine, 0)
        pltpu.async_copy(lrows.at[p], out_hbm.at[pl.ds(pl.multiple_of(base + b * BLK, 8), BLK)],
                         osem.at[p])

        @pl.when(b + 2 < nb)
        def _():
            pltpu.make_async_copy(
                lrows.at[p], out_hbm.at[pl.ds(pl.multiple_of(base + b * BLK, 8), BLK)],
                osem.at[p]).wait()
            start(b + 2, p)
        return carry
    lax.fori_loop(0, nb, body, 0)
    for t in (nb - 2, nb - 1):
        p = t % 2
        pltpu.make_async_copy(lrows.at[p],
                              out_hbm.at[pl.ds(pl.multiple_of(base + t * BLK, 8), BLK)],
                              osem.at[p]).wait()
  return _pair_mean


def _pair_mean(l_idx, r_idx, emb):
    if "pair" not in _agg_cache:
        _agg_cache["pair"] = _make_pair_mean()
    return _agg_cache["pair"](l_idx, r_idx, emb)


def _enc_body(x_ref, We1, be1, We2, be2, Wz, bz, e1_o, e2_o, z_o):
    h1 = jax.nn.relu(
        jnp.dot(x_ref[...], We1[...], preferred_element_type=jnp.float32)
        + be1[...])
    e1_o[...] = h1
    h2 = jax.nn.relu(
        jnp.dot(h1, We2[...], preferred_element_type=jnp.float32) + be2[...])
    e2_o[...] = h2
    z_o[...] = jnp.dot(h2, Wz[...], preferred_element_type=jnp.float32) \
        + bz[...]


def _dec_body(z_ref, Wd1, bd1, Wd2, bd2, Wxb, bxb, xbar_o):
    d1 = jax.nn.relu(
        jnp.dot(z_ref[...], Wd1[...], preferred_element_type=jnp.float32)
        + bd1[...])
    d2 = jax.nn.relu(
        jnp.dot(d1, Wd2[...], preferred_element_type=jnp.float32) + bd2[...])
    xbar_o[...] = jnp.dot(d2, Wxb[...],
                          preferred_element_type=jnp.float32) + bxb[...]


def _g1_body(aggx_ref, e1_ref, Wg1, Wg2, sup2_o):
    t = jnp.dot(aggx_ref[0], Wg1[0], preferred_element_type=jnp.float32)
    for cidx in range(1, 14):
        t = t + jnp.dot(aggx_ref[cidx], Wg1[cidx],
                        preferred_element_type=jnp.float32)
    mix = 0.5 * jax.nn.relu(t) + 0.5 * e1_ref[...]
    sup2_o[...] = jnp.dot(mix, Wg2[...], preferred_element_type=jnp.float32)


def _g3_body(agg2_ref, e2_ref, Wg3, sup3_o):
    acc = None
    for cidx in range(2):
        mix = 0.5 * jax.nn.relu(agg2_ref[cidx]) \
            + 0.5 * e2_ref[:, cidx * 128:(cidx + 1) * 128]
        d = jnp.dot(mix, Wg3[cidx], preferred_element_type=jnp.float32)
        acc = d if acc is None else acc + d
    sup3_o[...] = acc


def _att_body(agg3_ref, z_ref, Wa1, ba1, Wa2, emb1_o, beta_o):
    h3 = agg3_ref[0] + agg3_ref[1]
    z = z_ref[...]
    t3 = jnp.tanh(jnp.dot(h3, Wa1[...], preferred_element_type=jnp.float32)
                  + ba1[...])
    tz = jnp.tanh(jnp.dot(z, Wa1[...], preferred_element_type=jnp.float32)
                  + ba1[...])
    s3 = jnp.sum(t3 * Wa2[...], axis=1, keepdims=True)
    sz = jnp.sum(tz * Wa2[...], axis=1, keepdims=True)
    m = jnp.maximum(s3, sz)
    e3 = jnp.exp(s3 - m)
    ez = jnp.exp(sz - m)
    inv = 1.0 / (e3 + ez)
    b3 = e3 * inv
    bz = ez * inv
    emb1_o[...] = b3 * h3 + bz * z
    beta_o[...] = jnp.concatenate([b3, bz], axis=1)


def _mm_specs(shapes):
    return [pl.BlockSpec(s, lambda i, _n=len(s): (0,) * _n) for s in shapes]


def kernel(x, edge_index, edge_weight, ddi_pairs, labels, params):
    p = params
    f32 = jnp.float32

    We1 = jnp.pad(p['We1'], ((0, 0), (0, 48)))
    be1 = jnp.pad(p['be1'], (0, 48)).reshape(1, 2048)
    We2 = jnp.pad(p['We2'], ((0, 48), (0, 0)))
    be2 = p['be2'].reshape(1, 256)
    Wz = p['Wz']
    bz = p['bz'].reshape(1, 128)
    Wd1 = p['Wd1']
    bd1 = p['bd1'].reshape(1, 256)
    Wd2 = jnp.pad(p['Wd2'], ((0, 0), (0, 48)))
    bd2 = jnp.pad(p['bd2'], (0, 48)).reshape(1, 2048)
    Wxb = jnp.pad(p['Wxb'], ((0, 48), (0, 0)))
    bxb = p['bxb'].reshape(1, 1716)
    Wg1 = jnp.pad(p['Wg1'], ((0, 76), (0, 48))).reshape(14, 128, 2048)
    Wg2 = jnp.pad(p['Wg2'], ((0, 48), (0, 0)))
    Wg3 = p['Wg3'].reshape(2, 128, 128)
    Wa1 = p['Wa1']
    ba1 = p['ba1'].reshape(1, 128)
    Wa2 = p['Wa2'].reshape(1, 128)

    src = jnp.pad(edge_index[0].astype(jnp.int32), (0, E_PAD - E))
    dst_f = jnp.pad(edge_index[1].astype(jnp.int32), (0, E_PAD - E))
    ew = jnp.pad(edge_weight, (0, E_PAD - E))

    xT = jnp.pad(x, ((0, 0), (0, 76))).reshape(N, 14, 128) \
        .transpose(1, 0, 2).reshape(14 * N, 128)

    l_idx = jnp.pad(ddi_pairs[:, 0], (0, P_PAD - P))
    r_idx = jnp.pad(ddi_pairs[:, 1], (0, P_PAD - P))

    enc_h1, enc_h2, z = pl.pallas_call(
        _enc_body,
        grid=(GRID_M,),
        in_specs=[pl.BlockSpec((BM, 1716), lambda i: (i, 0))]
        + _mm_specs([(1716, 2048), (1, 2048), (2048, 256), (1, 256),
                     (256, 128), (1, 128)]),
        out_specs=[pl.BlockSpec((BM, 2048), lambda i: (i, 0)),
                   pl.BlockSpec((BM, 256), lambda i: (i, 0)),
                   pl.BlockSpec((BM, 128), lambda i: (i, 0))],
        out_shape=[jax.ShapeDtypeStruct((N, 2048), f32),
                   jax.ShapeDtypeStruct((N, 256), f32),
                   jax.ShapeDtypeStruct((N, 128), f32)],
    )(x, We1, be1, We2, be2, Wz, bz)

    x_bar = pl.pallas_call(
        _dec_body,
        grid=(GRID_M,),
        in_specs=[pl.BlockSpec((BM, 128), lambda i: (i, 0))]
        + _mm_specs([(128, 256), (1, 256), (256, 2048), (1, 2048),
                     (2048, 1716), (1, 1716)]),
        out_specs=pl.BlockSpec((BM, 1716), lambda i: (i, 0)),
        out_shape=jax.ShapeDtypeStruct((N, 1716), f32),
    )(z, Wd1, bd1, Wd2, bd2, Wxb, bxb)

    aggx = _agg(14)(src, dst_f, ew, xT).reshape(14, N, 128)

    sup2 = pl.pallas_call(
        _g1_body,
        grid=(GRID_M,),
        in_specs=[pl.BlockSpec((14, BM, 128), lambda i: (0, i, 0)),
                  pl.BlockSpec((BM, 2048), lambda i: (i, 0))]
        + _mm_specs([(14, 128, 2048), (2048, 256)]),
        out_specs=pl.BlockSpec((BM, 256), lambda i: (i, 0)),
        out_shape=jax.ShapeDtypeStruct((N, 256), f32),
    )(aggx, enc_h1, Wg1, Wg2)

    sup2T = sup2.reshape(N, 2, 128).transpose(1, 0, 2).reshape(2 * N, 128)
    agg2 = _agg(2)(src, dst_f, ew, sup2T).reshape(2, N, 128)

    sup3 = pl.pallas_call(
        _g3_body,
        grid=(GRID_M,),
        in_specs=[pl.BlockSpec((2, BM, 128), lambda i: (0, i, 0)),
                  pl.BlockSpec((BM, 256), lambda i: (i, 0))]
        + _mm_specs([(2, 128, 128)]),
        out_specs=pl.BlockSpec((BM, 128), lambda i: (i, 0)),
        out_shape=jax.ShapeDtypeStruct((N, 128), f32),
    )(agg2, enc_h2, Wg3)

    agg3 = _agg(1)(src, dst_f, ew, sup3).reshape(2, N, 128)

    emb1, beta2 = pl.pallas_call(
        _att_body,
        grid=(GRID_M,),
        in_specs=[pl.BlockSpec((2, BM, 128), lambda i: (0, i, 0)),
                  pl.BlockSpec((BM, 128), lambda i: (i, 0))]
        + _mm_specs([(128, 128), (1, 128), (1, 128)]),
        out_specs=[pl.BlockSpec((BM, 128), lambda i: (i, 0)),
                   pl.BlockSpec((BM, 2), lambda i: (i, 0))],
        out_shape=[jax.ShapeDtypeStruct((N, 128), f32),
                   jax.ShapeDtypeStruct((N, 2), f32)],
    )(agg3, z, Wa1, ba1, Wa2)

    beta = beta2.reshape(N, 2, 1)

    Bfull = _pair_mean(l_idx, r_idx, emb1)
    C1 = Bfull[:NUM_TRAIN]
    C2 = Bfull[NUM_TRAIN:P]

    return (emb1, beta, x_bar, C1, C2, labels[:NUM_TRAIN], labels[NUM_TRAIN:])


# speedup vs baseline: 1.0219x; 1.0219x over previous
import functools

import jax
import jax.numpy as jnp
from jax import lax
from jax.experimental import pallas as pl
from jax.experimental.pallas import tpu as pltpu, tpu_sc as plsc

N = 10000
E = 160000
P = 100000
NUM_TRAIN = 80000

E_PAD = 163840
P_PAD = 102400
BLK = 80
BM = 400
GRID_M = N // BM

def _mesh():
    return plsc.VectorSubcoreMesh(core_axis_name="c", subcore_axis_name="s")


def _make_agg(C):
    split_edges = (C == 1)
    if split_edges:
        edges_per_worker = E_PAD // 32
        out_rows = 2 * N
    else:
        edges_per_worker = E_PAD // 16
        out_rows = C * N
    nb = edges_per_worker // BLK
    chunks_per_core = 1 if split_edges else C // 2
    zrows = 16
    nrowblocks = N // 80
    ER = 6

    @functools.partial(
        pl.kernel,
        out_type=jax.ShapeDtypeStruct((out_rows, 128), jnp.float32),
        mesh=_mesh(),
        scratch_types=[
            pltpu.VMEM((ER, BLK), jnp.int32),
            pltpu.VMEM((ER, BLK), jnp.int32),
            pltpu.VMEM((ER, BLK), jnp.float32),
            pltpu.VMEM((4, BLK), jnp.int32),
            pltpu.VMEM((4, BLK, 128), jnp.float32),
            pltpu.VMEM((zrows, 128), jnp.float32),
            pltpu.VMEM_SHARED((N, 128), jnp.float32),
            pltpu.SemaphoreType.DMA((ER,)),
            pltpu.SemaphoreType.DMA((ER,)),
            pltpu.SemaphoreType.DMA((ER,)),
            pltpu.SemaphoreType.DMA((4,)),
            pltpu.SemaphoreType.DMA((4,)),
        ],
    )
    def agg(src_hbm, dst_hbm, w_hbm, table_hbm, out_hbm,
            src_r, dst_r, w_r, idx_v, rows_v, zero_v, acc_sh,
            esrc, edst, ew_sem, gsem, ssem):
        c = lax.axis_index("c")
        s = lax.axis_index("s")
        if split_edges:
            ebase = (c * 16 + s) * edges_per_worker
        else:
            ebase = s * edges_per_worker
        ebase = pl.multiple_of(ebase, 128)
        for j in range(zrows):
            for k in range(8):
                zero_v[j, pl.ds(k * 16, 16)] = jnp.zeros((16,), jnp.float32)

        def estage(b):
            pe = lax.rem(b, ER)
            off = pl.multiple_of(ebase + b * BLK, 8)
            pltpu.async_copy(src_hbm.at[pl.ds(off, BLK)], src_r.at[pe],
                             esrc.at[pe])
            pltpu.async_copy(dst_hbm.at[pl.ds(off, BLK)], dst_r.at[pe],
                             edst.at[pe])
            pltpu.async_copy(w_hbm.at[pl.ds(off, BLK)], w_r.at[pe],
                             ew_sem.at[pe])

        def ewait(b):
            pe = lax.rem(b, ER)
            off = pl.multiple_of(ebase + b * BLK, 8)
            pltpu.make_async_copy(src_hbm.at[pl.ds(off, BLK)], src_r.at[pe],
                                  esrc.at[pe]).wait()
            pltpu.make_async_copy(dst_hbm.at[pl.ds(off, BLK)], dst_r.at[pe],
                                  edst.at[pe]).wait()
            pltpu.make_async_copy(w_hbm.at[pl.ds(off, BLK)], w_r.at[pe],
                                  ew_sem.at[pe]).wait()

        trips = (nrowblocks - 1 - s) // 16 + 1

        def run_chunk(chunk, out_base):
            def zbody(i, carry):
                off = pl.multiple_of((s + i * 16) * 80, 8)
                for rr in range(80 // zrows):
                    pltpu.sync_copy(zero_v,
                                    acc_sh.at[pl.ds(off + rr * zrows, zrows)])
                return carry
            lax.fori_loop(0, trips, zbody, 0)
            plsc.subcore_barrier()
            cbase = chunk * N

            def fill_idx(b, p):
                pe = lax.rem(b, ER)
                for k in range(BLK // 16):
                    idx_v[p, pl.ds(k * 16, 16)] = (
                        src_r[pe, pl.ds(k * 16, 16)] + cbase)

            def start_gather(b, p):
                fill_idx(b, p)
                pltpu.async_copy(table_hbm.at[idx_v.at[p]], rows_v.at[p],
                                 gsem.at[p])

            def scale(b, p):
                pe = lax.rem(b, ER)

                @plsc.parallel_loop(0, BLK // 16, unroll=2)
                def _(g):
                    w16 = w_r[pe, pl.ds(g * 16, 16)]
                    for i in range(16):
                        ws = w16.at[jnp.full((16,), i, jnp.int32)].get(
                            mode="promise_in_bounds")
                        vals = [rows_v[p, g * 16 + i, pl.ds(k * 16, 16)] * ws
                                for k in range(8)]
                        for k in range(8):
                            rows_v[p, g * 16 + i, pl.ds(k * 16, 16)] = vals[k]

            def start_scatter(b, p):
                pass

            def wait_scatter(p):
                pass

            for b0 in range(5):
                estage(b0)
            for b0 in range(2):
                ewait(b0)
                start_gather(b0, b0)

            def body(b, carry):
                p = lax.rem(b, 4)

                @pl.when(b + 5 < nb)
                def _():
                    estage(b + 5)
                pltpu.make_async_copy(table_hbm.at[idx_v.at[p]],
                                      rows_v.at[p], gsem.at[p]).wait()

                @pl.when(b + 2 < nb)
                def _():
                    q = lax.rem(b + 2, 4)
                    ewait(b + 2)

                    @pl.when(b >= 2)
                    def _():
                        wait_scatter(q)
                    start_gather(b + 2, q)
                scale(b, p)
                start_scatter(b, p)
                return carry
            lax.fori_loop(0, nb, body, 0)
            for t in (nb - 4, nb - 3, nb - 2, nb - 1):
                wait_scatter(t % 4)
            plsc.subcore_barrier()

            def obody(i, carry):
                off = pl.multiple_of((s + i * 16) * 80, 8)
                pltpu.sync_copy(
                    acc_sh.at[pl.ds(off, 80)],
                    out_hbm.at[pl.ds(pl.multiple_of(out_base + off, 8), 80)])
                return carry
            lax.fori_loop(0, trips, obody, 0)
            plsc.subcore_barrier()

        if split_edges:
            run_chunk(0, c * N)
        else:
            def chunk_body(kk, carry):
                chunk = kk * 2 + c
                run_chunk(chunk, chunk * N)
                return carry
            lax.fori_loop(0, chunks_per_core, chunk_body, 0)

    return agg


_agg_cache = {}


def _agg(C):
    if C not in _agg_cache:
        _agg_cache[C] = _make_agg(C)
    return _agg_cache[C]


def _make_pair_mean():
  @functools.partial(
    pl.kernel,
    out_type=jax.ShapeDtypeStruct((P_PAD, 128), jnp.float32),
    mesh=_mesh(),
    scratch_types=[
        pltpu.VMEM((P_PAD // 32,), jnp.int32),
        pltpu.VMEM((P_PAD // 32,), jnp.int32),
        pltpu.VMEM((2, BLK, 128), jnp.float32),
        pltpu.VMEM((2, BLK, 128), jnp.float32),
        pltpu.SemaphoreType.DMA((2,)),
        pltpu.SemaphoreType.DMA((2,)),
        pltpu.SemaphoreType.DMA((2,)),
    ],
  )
  def _pair_mean(l_hbm, r_hbm, emb_hbm, out_hbm,
               l_v, r_v, lrows, rrows, lsem, rsem, osem):
    c = lax.axis_index("c")
    s = lax.axis_index("s")
    wid = c * 16 + s
    per_w = P_PAD // 32
    nb = per_w // BLK
    base = pl.multiple_of(wid * per_w, 128)
    pltpu.sync_copy(l_hbm.at[pl.ds(base, per_w)], l_v)
    pltpu.sync_copy(r_hbm.at[pl.ds(base, per_w)], r_v)

    def start(b, p):
        pltpu.async_copy(emb_hbm.at[l_v.at[pl.ds(b * BLK, BLK)]],
                         lrows.at[p], lsem.at[p])
        pltpu.async_copy(emb_hbm.at[r_v.at[pl.ds(b * BLK, BLK)]],
                         rrows.at[p], rsem.at[p])

    start(0, 0)
    start(1, 1)

    def body(b, carry):
        p = lax.rem(b, 2)
        pltpu.make_async_copy(emb_hbm.at[l_v.at[pl.ds(b * BLK, BLK)]],
                              lrows.at[p], lsem.at[p]).wait()
        pltpu.make_async_copy(emb_hbm.at[r_v.at[pl.ds(b * BLK, BLK)]],
                              rrows.at[p], rsem.at[p]).wait()

        def combine(e, carry2):
            for k in range(8):
                lrows[p, e, pl.ds(k * 16, 16)] = (
                    lrows[p, e, pl.ds(k * 16, 16)]
                    + rrows[p, e, pl.ds(k * 16, 16)]) * 0.5
            return carry2
        lax.fori_loop(0, BLK, combine, 0)
        pltpu.async_copy(lrows.at[p], out_hbm.at[pl.ds(pl.multiple_of(base + b * BLK, 8), BLK)],
                         osem.at[p])

        @pl.when(b + 2 < nb)
        def _():
            pltpu.make_async_copy(
                lrows.at[p], out_hbm.at[pl.ds(pl.multiple_of(base + b * BLK, 8), BLK)],
                osem.at[p]).wait()
            start(b + 2, p)
        return carry
    lax.fori_loop(0, nb, body, 0)
    for t in (nb - 2, nb - 1):
        p = t % 2
        pltpu.make_async_copy(lrows.at[p],
                              out_hbm.at[pl.ds(pl.multiple_of(base + t * BLK, 8), BLK)],
                              osem.at[p]).wait()
  return _pair_mean


def _pair_mean(l_idx, r_idx, emb):
    if "pair" not in _agg_cache:
        _agg_cache["pair"] = _make_pair_mean()
    return _agg_cache["pair"](l_idx, r_idx, emb)


def _enc_body(x_ref, We1, be1, We2, be2, Wz, bz, e1_o, e2_o, z_o):
    h1 = jax.nn.relu(
        jnp.dot(x_ref[...], We1[...], preferred_element_type=jnp.float32)
        + be1[...])
    e1_o[...] = h1
    h2 = jax.nn.relu(
        jnp.dot(h1, We2[...], preferred_element_type=jnp.float32) + be2[...])
    e2_o[...] = h2
    z_o[...] = jnp.dot(h2, Wz[...], preferred_element_type=jnp.float32) \
        + bz[...]


def _dec_body(z_ref, Wd1, bd1, Wd2, bd2, Wxb, bxb, xbar_o):
    d1 = jax.nn.relu(
        jnp.dot(z_ref[...], Wd1[...], preferred_element_type=jnp.float32)
        + bd1[...])
    d2 = jax.nn.relu(
        jnp.dot(d1, Wd2[...], preferred_element_type=jnp.float32) + bd2[...])
    xbar_o[...] = jnp.dot(d2, Wxb[...],
                          preferred_element_type=jnp.float32) + bxb[...]


def _g1_body(aggx_ref, e1_ref, Wg1, Wg2, sup2_o):
    t = jnp.dot(aggx_ref[0], Wg1[0], preferred_element_type=jnp.float32)
    for cidx in range(1, 14):
        t = t + jnp.dot(aggx_ref[cidx], Wg1[cidx],
                        preferred_element_type=jnp.float32)
    mix = 0.5 * jax.nn.relu(t) + 0.5 * e1_ref[...]
    sup2_o[...] = jnp.dot(mix, Wg2[...], preferred_element_type=jnp.float32)


def _g3_body(agg2_ref, e2_ref, Wg3, sup3_o):
    acc = None
    for cidx in range(2):
        mix = 0.5 * jax.nn.relu(agg2_ref[cidx]) \
            + 0.5 * e2_ref[:, cidx * 128:(cidx + 1) * 128]
        d = jnp.dot(mix, Wg3[cidx], preferred_element_type=jnp.float32)
        acc = d if acc is None else acc + d
    sup3_o[...] = acc


def _att_body(agg3_ref, z_ref, Wa1, ba1, Wa2, emb1_o, beta_o):
    h3 = agg3_ref[0] + agg3_ref[1]
    z = z_ref[...]
    t3 = jnp.tanh(jnp.dot(h3, Wa1[...], preferred_element_type=jnp.float32)
                  + ba1[...])
    tz = jnp.tanh(jnp.dot(z, Wa1[...], preferred_element_type=jnp.float32)
                  + ba1[...])
    s3 = jnp.sum(t3 * Wa2[...], axis=1, keepdims=True)
    sz = jnp.sum(tz * Wa2[...], axis=1, keepdims=True)
    m = jnp.maximum(s3, sz)
    e3 = jnp.exp(s3 - m)
    ez = jnp.exp(sz - m)
    inv = 1.0 / (e3 + ez)
    b3 = e3 * inv
    bz = ez * inv
    emb1_o[...] = b3 * h3 + bz * z
    beta_o[...] = jnp.concatenate([b3, bz], axis=1)


def _mm_specs(shapes):
    return [pl.BlockSpec(s, lambda i, _n=len(s): (0,) * _n) for s in shapes]


def kernel(x, edge_index, edge_weight, ddi_pairs, labels, params):
    p = params
    f32 = jnp.float32

    We1 = jnp.pad(p['We1'], ((0, 0), (0, 48)))
    be1 = jnp.pad(p['be1'], (0, 48)).reshape(1, 2048)
    We2 = jnp.pad(p['We2'], ((0, 48), (0, 0)))
    be2 = p['be2'].reshape(1, 256)
    Wz = p['Wz']
    bz = p['bz'].reshape(1, 128)
    Wd1 = p['Wd1']
    bd1 = p['bd1'].reshape(1, 256)
    Wd2 = jnp.pad(p['Wd2'], ((0, 0), (0, 48)))
    bd2 = jnp.pad(p['bd2'], (0, 48)).reshape(1, 2048)
    Wxb = jnp.pad(p['Wxb'], ((0, 48), (0, 0)))
    bxb = p['bxb'].reshape(1, 1716)
    Wg1 = jnp.pad(p['Wg1'], ((0, 76), (0, 48))).reshape(14, 128, 2048)
    Wg2 = jnp.pad(p['Wg2'], ((0, 48), (0, 0)))
    Wg3 = p['Wg3'].reshape(2, 128, 128)
    Wa1 = p['Wa1']
    ba1 = p['ba1'].reshape(1, 128)
    Wa2 = p['Wa2'].reshape(1, 128)

    src = jnp.pad(edge_index[0].astype(jnp.int32), (0, E_PAD - E))
    dst_f = jnp.pad(edge_index[1].astype(jnp.int32), (0, E_PAD - E))
    ew = jnp.pad(edge_weight, (0, E_PAD - E))

    xT = jnp.pad(x, ((0, 0), (0, 76))).reshape(N, 14, 128) \
        .transpose(1, 0, 2).reshape(14 * N, 128)

    l_idx = jnp.pad(ddi_pairs[:, 0], (0, P_PAD - P))
    r_idx = jnp.pad(ddi_pairs[:, 1], (0, P_PAD - P))

    enc_h1, enc_h2, z = pl.pallas_call(
        _enc_body,
        grid=(GRID_M,),
        in_specs=[pl.BlockSpec((BM, 1716), lambda i: (i, 0))]
        + _mm_specs([(1716, 2048), (1, 2048), (2048, 256), (1, 256),
                     (256, 128), (1, 128)]),
        out_specs=[pl.BlockSpec((BM, 2048), lambda i: (i, 0)),
                   pl.BlockSpec((BM, 256), lambda i: (i, 0)),
                   pl.BlockSpec((BM, 128), lambda i: (i, 0))],
        out_shape=[jax.ShapeDtypeStruct((N, 2048), f32),
                   jax.ShapeDtypeStruct((N, 256), f32),
                   jax.ShapeDtypeStruct((N, 128), f32)],
    )(x, We1, be1, We2, be2, Wz, bz)

    x_bar = pl.pallas_call(
        _dec_body,
        grid=(GRID_M,),
        in_specs=[pl.BlockSpec((BM, 128), lambda i: (i, 0))]
        + _mm_specs([(128, 256), (1, 256), (256, 2048), (1, 2048),
                     (2048, 1716), (1, 1716)]),
        out_specs=pl.BlockSpec((BM, 1716), lambda i: (i, 0)),
        out_shape=jax.ShapeDtypeStruct((N, 1716), f32),
    )(z, Wd1, bd1, Wd2, bd2, Wxb, bxb)

    aggx = _agg(14)(src, dst_f, ew, xT).reshape(14, N, 128)

    sup2 = pl.pallas_call(
        _g1_body,
        grid=(GRID_M,),
        in_specs=[pl.BlockSpec((14, BM, 128), lambda i: (0, i, 0)),
                  pl.BlockSpec((BM, 2048), lambda i: (i, 0))]
        + _mm_specs([(14, 128, 2048), (2048, 256)]),
        out_specs=pl.BlockSpec((BM, 256), lambda i: (i, 0)),
        out_shape=jax.ShapeDtypeStruct((N, 256), f32),
    )(aggx, enc_h1, Wg1, Wg2)

    sup2T = sup2.reshape(N, 2, 128).transpose(1, 0, 2).reshape(2 * N, 128)
    agg2 = _agg(2)(src, dst_f, ew, sup2T).reshape(2, N, 128)

    sup3 = pl.pallas_call(
        _g3_body,
        grid=(GRID_M,),
        in_specs=[pl.BlockSpec((2, BM, 128), lambda i: (0, i, 0)),
                  pl.BlockSpec((BM, 256), lambda i: (i, 0))]
        + _mm_specs([(2, 128, 128)]),
        out_specs=pl.BlockSpec((BM, 128), lambda i: (i, 0)),
        out_shape=jax.ShapeDtypeStruct((N, 128), f32),
    )(agg2, enc_h2, Wg3)

    agg3 = _agg(1)(src, dst_f, ew, sup3).reshape(2, N, 128)

    emb1, beta2 = pl.pallas_call(
        _att_body,
        grid=(GRID_M,),
        in_specs=[pl.BlockSpec((2, BM, 128), lambda i: (0, i, 0)),
                  pl.BlockSpec((BM, 128), lambda i: (i, 0))]
        + _mm_specs([(128, 128), (1, 128), (1, 128)]),
        out_specs=[pl.BlockSpec((BM, 128), lambda i: (i, 0)),
                   pl.BlockSpec((BM, 2), lambda i: (i, 0))],
        out_shape=[jax.ShapeDtypeStruct((N, 128), f32),
                   jax.ShapeDtypeStruct((N, 2), f32)],
    )(agg3, z, Wa1, ba1, Wa2)

    beta = beta2.reshape(N, 2, 1)

    Bfull = _pair_mean(l_idx, r_idx, emb1)
    C1 = Bfull[:NUM_TRAIN]
    C2 = Bfull[NUM_TRAIN:P]

    return (emb1, beta, x_bar, C1, C2, labels[:NUM_TRAIN], labels[NUM_TRAIN:])


# speedup vs baseline: 2.1434x; 2.0975x over previous
import functools

import jax
import jax.numpy as jnp
from jax import lax
from jax.experimental import pallas as pl
from jax.experimental.pallas import tpu as pltpu, tpu_sc as plsc

N = 10000
E = 160000
P = 100000
NUM_TRAIN = 80000

E_PAD = 163840
P_PAD = 102400
BLK = 80
BM = 400
GRID_M = N // BM

def _mesh():
    return plsc.VectorSubcoreMesh(core_axis_name="c", subcore_axis_name="s")


def _make_agg(C):
    split_edges = (C == 1)
    if split_edges:
        edges_per_worker = E_PAD // 32
        out_rows = 2 * N
    else:
        edges_per_worker = E_PAD // 16
        out_rows = C * N
    nb = edges_per_worker // BLK
    chunks_per_core = 1 if split_edges else C // 2
    zrows = 16
    nrowblocks = N // 80
    ER = 6

    @functools.partial(
        pl.kernel,
        out_type=jax.ShapeDtypeStruct((out_rows, 128), jnp.float32),
        mesh=_mesh(),
        scratch_types=[
            pltpu.VMEM((ER, BLK), jnp.int32),
            pltpu.VMEM((ER, BLK), jnp.int32),
            pltpu.VMEM((ER, BLK), jnp.float32),
            pltpu.VMEM((4, BLK), jnp.int32),
            pltpu.VMEM((4, BLK, 128), jnp.float32),
            pltpu.VMEM((zrows, 128), jnp.float32),
            pltpu.VMEM_SHARED((N, 128), jnp.float32),
            pltpu.SemaphoreType.DMA((ER,)),
            pltpu.SemaphoreType.DMA((ER,)),
            pltpu.SemaphoreType.DMA((ER,)),
            pltpu.SemaphoreType.DMA((4,)),
            pltpu.SemaphoreType.DMA((4,)),
        ],
    )
    def agg(src_hbm, dst_hbm, w_hbm, table_hbm, out_hbm,
            src_r, dst_r, w_r, idx_v, rows_v, zero_v, acc_sh,
            esrc, edst, ew_sem, gsem, ssem):
        c = lax.axis_index("c")
        s = lax.axis_index("s")
        if split_edges:
            ebase = (c * 16 + s) * edges_per_worker
        else:
            ebase = s * edges_per_worker
        ebase = pl.multiple_of(ebase, 128)
        for j in range(zrows):
            for k in range(8):
                zero_v[j, pl.ds(k * 16, 16)] = jnp.zeros((16,), jnp.float32)

        def estage(b):
            pe = lax.rem(b, ER)
            off = pl.multiple_of(ebase + b * BLK, 8)
            pltpu.async_copy(src_hbm.at[pl.ds(off, BLK)], src_r.at[pe],
                             esrc.at[pe])
            pltpu.async_copy(dst_hbm.at[pl.ds(off, BLK)], dst_r.at[pe],
                             edst.at[pe])
            pltpu.async_copy(w_hbm.at[pl.ds(off, BLK)], w_r.at[pe],
                             ew_sem.at[pe])

        def ewait(b):
            pe = lax.rem(b, ER)
            off = pl.multiple_of(ebase + b * BLK, 8)
            pltpu.make_async_copy(src_hbm.at[pl.ds(off, BLK)], src_r.at[pe],
                                  esrc.at[pe]).wait()
            pltpu.make_async_copy(dst_hbm.at[pl.ds(off, BLK)], dst_r.at[pe],
                                  edst.at[pe]).wait()
            pltpu.make_async_copy(w_hbm.at[pl.ds(off, BLK)], w_r.at[pe],
                                  ew_sem.at[pe]).wait()

        trips = (nrowblocks - 1 - s) // 16 + 1

        def run_chunk(chunk, out_base):
            def zbody(i, carry):
                off = pl.multiple_of((s + i * 16) * 80, 8)
                for rr in range(80 // zrows):
                    pltpu.sync_copy(zero_v,
                                    acc_sh.at[pl.ds(off + rr * zrows, zrows)])
                return carry
            lax.fori_loop(0, trips, zbody, 0)
            plsc.subcore_barrier()
            cbase = chunk * N

            def fill_idx(b, p):
                pe = lax.rem(b, ER)
                for k in range(BLK // 16):
                    idx_v[p, pl.ds(k * 16, 16)] = (
                        src_r[pe, pl.ds(k * 16, 16)] + cbase)

            def start_gather(b, p):
                fill_idx(b, p)

            def scale(b, p):
                pe = lax.rem(b, ER)

                @plsc.parallel_loop(0, BLK // 16, unroll=2)
                def _(g):
                    w16 = w_r[pe, pl.ds(g * 16, 16)]
                    for i in range(16):
                        ws = w16.at[jnp.full((16,), i, jnp.int32)].get(
                            mode="promise_in_bounds")
                        vals = [rows_v[p, g * 16 + i, pl.ds(k * 16, 16)] * ws
                                for k in range(8)]
                        for k in range(8):
                            rows_v[p, g * 16 + i, pl.ds(k * 16, 16)] = vals[k]

            def start_scatter(b, p):
                pe = lax.rem(b, ER)
                for g in range(BLK // 16):
                    dvec = dst_r[pe, pl.ds(g * 16, 16)]
                    pltpu.async_copy(rows_v.at[p, pl.ds(g * 16, 16)],
                                     acc_sh.at[dvec], ssem.at[p], add=True)

            def wait_scatter(p):
                dvec = dst_r[0, pl.ds(0, 16)]
                for g in range(BLK // 16):
                    pltpu.make_async_copy(rows_v.at[p, pl.ds(g * 16, 16)],
                                          acc_sh.at[dvec],
                                          ssem.at[p]).wait()

            for b0 in range(5):
                estage(b0)
            for b0 in range(2):
                ewait(b0)
                start_gather(b0, b0)

            def body(b, carry):
                p = lax.rem(b, 4)

                @pl.when(b + 5 < nb)
                def _():
                    estage(b + 5)

                @pl.when(b + 2 < nb)
                def _():
                    q = lax.rem(b + 2, 4)
                    ewait(b + 2)

                    @pl.when(b >= 2)
                    def _():
                        wait_scatter(q)
                    start_gather(b + 2, q)
                scale(b, p)
                start_scatter(b, p)
                return carry
            lax.fori_loop(0, nb, body, 0)
            for t in (nb - 4, nb - 3, nb - 2, nb - 1):
                wait_scatter(t % 4)
            plsc.subcore_barrier()

            def obody(i, carry):
                off = pl.multiple_of((s + i * 16) * 80, 8)
                pltpu.sync_copy(
                    acc_sh.at[pl.ds(off, 80)],
                    out_hbm.at[pl.ds(pl.multiple_of(out_base + off, 8), 80)])
                return carry
            lax.fori_loop(0, trips, obody, 0)
            plsc.subcore_barrier()

        if split_edges:
            run_chunk(0, c * N)
        else:
            def chunk_body(kk, carry):
                chunk = kk * 2 + c
                run_chunk(chunk, chunk * N)
                return carry
            lax.fori_loop(0, chunks_per_core, chunk_body, 0)

    return agg


_agg_cache = {}


def _agg(C):
    if C not in _agg_cache:
        _agg_cache[C] = _make_agg(C)
    return _agg_cache[C]


def _make_pair_mean():
  @functools.partial(
    pl.kernel,
    out_type=jax.ShapeDtypeStruct((P_PAD, 128), jnp.float32),
    mesh=_mesh(),
    scratch_types=[
        pltpu.VMEM((P_PAD // 32,), jnp.int32),
        pltpu.VMEM((P_PAD // 32,), jnp.int32),
        pltpu.VMEM((2, BLK, 128), jnp.float32),
        pltpu.VMEM((2, BLK, 128), jnp.float32),
        pltpu.SemaphoreType.DMA((2,)),
        pltpu.SemaphoreType.DMA((2,)),
        pltpu.SemaphoreType.DMA((2,)),
    ],
  )
  def _pair_mean(l_hbm, r_hbm, emb_hbm, out_hbm,
               l_v, r_v, lrows, rrows, lsem, rsem, osem):
    c = lax.axis_index("c")
    s = lax.axis_index("s")
    wid = c * 16 + s
    per_w = P_PAD // 32
    nb = per_w // BLK
    base = pl.multiple_of(wid * per_w, 128)
    pltpu.sync_copy(l_hbm.at[pl.ds(base, per_w)], l_v)
    pltpu.sync_copy(r_hbm.at[pl.ds(base, per_w)], r_v)

    def start(b, p):
        pltpu.async_copy(emb_hbm.at[l_v.at[pl.ds(b * BLK, BLK)]],
                         lrows.at[p], lsem.at[p])
        pltpu.async_copy(emb_hbm.at[r_v.at[pl.ds(b * BLK, BLK)]],
                         rrows.at[p], rsem.at[p])

    start(0, 0)
    start(1, 1)

    def body(b, carry):
        p = lax.rem(b, 2)
        pltpu.make_async_copy(emb_hbm.at[l_v.at[pl.ds(b * BLK, BLK)]],
                              lrows.at[p], lsem.at[p]).wait()
        pltpu.make_async_copy(emb_hbm.at[r_v.at[pl.ds(b * BLK, BLK)]],
                              rrows.at[p], rsem.at[p]).wait()

        def combine(e, carry2):
            for k in range(8):
                lrows[p, e, pl.ds(k * 16, 16)] = (
                    lrows[p, e, pl.ds(k * 16, 16)]
                    + rrows[p, e, pl.ds(k * 16, 16)]) * 0.5
            return carry2
        lax.fori_loop(0, BLK, combine, 0)
        pltpu.async_copy(lrows.at[p], out_hbm.at[pl.ds(pl.multiple_of(base + b * BLK, 8), BLK)],
                         osem.at[p])

        @pl.when(b + 2 < nb)
        def _():
            pltpu.make_async_copy(
                lrows.at[p], out_hbm.at[pl.ds(pl.multiple_of(base + b * BLK, 8), BLK)],
                osem.at[p]).wait()
            start(b + 2, p)
        return carry
    lax.fori_loop(0, nb, body, 0)
    for t in (nb - 2, nb - 1):
        p = t % 2
        pltpu.make_async_copy(lrows.at[p],
                              out_hbm.at[pl.ds(pl.multiple_of(base + t * BLK, 8), BLK)],
                              osem.at[p]).wait()
  return _pair_mean


def _pair_mean(l_idx, r_idx, emb):
    if "pair" not in _agg_cache:
        _agg_cache["pair"] = _make_pair_mean()
    return _agg_cache["pair"](l_idx, r_idx, emb)


def _enc_body(x_ref, We1, be1, We2, be2, Wz, bz, e1_o, e2_o, z_o):
    h1 = jax.nn.relu(
        jnp.dot(x_ref[...], We1[...], preferred_element_type=jnp.float32)
        + be1[...])
    e1_o[...] = h1
    h2 = jax.nn.relu(
        jnp.dot(h1, We2[...], preferred_element_type=jnp.float32) + be2[...])
    e2_o[...] = h2
    z_o[...] = jnp.dot(h2, Wz[...], preferred_element_type=jnp.float32) \
        + bz[...]


def _dec_body(z_ref, Wd1, bd1, Wd2, bd2, Wxb, bxb, xbar_o):
    d1 = jax.nn.relu(
        jnp.dot(z_ref[...], Wd1[...], preferred_element_type=jnp.float32)
        + bd1[...])
    d2 = jax.nn.relu(
        jnp.dot(d1, Wd2[...], preferred_element_type=jnp.float32) + bd2[...])
    xbar_o[...] = jnp.dot(d2, Wxb[...],
                          preferred_element_type=jnp.float32) + bxb[...]


def _g1_body(aggx_ref, e1_ref, Wg1, Wg2, sup2_o):
    t = jnp.dot(aggx_ref[0], Wg1[0], preferred_element_type=jnp.float32)
    for cidx in range(1, 14):
        t = t + jnp.dot(aggx_ref[cidx], Wg1[cidx],
                        preferred_element_type=jnp.float32)
    mix = 0.5 * jax.nn.relu(t) + 0.5 * e1_ref[...]
    sup2_o[...] = jnp.dot(mix, Wg2[...], preferred_element_type=jnp.float32)


def _g3_body(agg2_ref, e2_ref, Wg3, sup3_o):
    acc = None
    for cidx in range(2):
        mix = 0.5 * jax.nn.relu(agg2_ref[cidx]) \
            + 0.5 * e2_ref[:, cidx * 128:(cidx + 1) * 128]
        d = jnp.dot(mix, Wg3[cidx], preferred_element_type=jnp.float32)
        acc = d if acc is None else acc + d
    sup3_o[...] = acc


def _att_body(agg3_ref, z_ref, Wa1, ba1, Wa2, emb1_o, beta_o):
    h3 = agg3_ref[0] + agg3_ref[1]
    z = z_ref[...]
    t3 = jnp.tanh(jnp.dot(h3, Wa1[...], preferred_element_type=jnp.float32)
                  + ba1[...])
    tz = jnp.tanh(jnp.dot(z, Wa1[...], preferred_element_type=jnp.float32)
                  + ba1[...])
    s3 = jnp.sum(t3 * Wa2[...], axis=1, keepdims=True)
    sz = jnp.sum(tz * Wa2[...], axis=1, keepdims=True)
    m = jnp.maximum(s3, sz)
    e3 = jnp.exp(s3 - m)
    ez = jnp.exp(sz - m)
    inv = 1.0 / (e3 + ez)
    b3 = e3 * inv
    bz = ez * inv
    emb1_o[...] = b3 * h3 + bz * z
    beta_o[...] = jnp.concatenate([b3, bz], axis=1)


def _mm_specs(shapes):
    return [pl.BlockSpec(s, lambda i, _n=len(s): (0,) * _n) for s in shapes]


def kernel(x, edge_index, edge_weight, ddi_pairs, labels, params):
    p = params
    f32 = jnp.float32

    We1 = jnp.pad(p['We1'], ((0, 0), (0, 48)))
    be1 = jnp.pad(p['be1'], (0, 48)).reshape(1, 2048)
    We2 = jnp.pad(p['We2'], ((0, 48), (0, 0)))
    be2 = p['be2'].reshape(1, 256)
    Wz = p['Wz']
    bz = p['bz'].reshape(1, 128)
    Wd1 = p['Wd1']
    bd1 = p['bd1'].reshape(1, 256)
    Wd2 = jnp.pad(p['Wd2'], ((0, 0), (0, 48)))
    bd2 = jnp.pad(p['bd2'], (0, 48)).reshape(1, 2048)
    Wxb = jnp.pad(p['Wxb'], ((0, 48), (0, 0)))
    bxb = p['bxb'].reshape(1, 1716)
    Wg1 = jnp.pad(p['Wg1'], ((0, 76), (0, 48))).reshape(14, 128, 2048)
    Wg2 = jnp.pad(p['Wg2'], ((0, 48), (0, 0)))
    Wg3 = p['Wg3'].reshape(2, 128, 128)
    Wa1 = p['Wa1']
    ba1 = p['ba1'].reshape(1, 128)
    Wa2 = p['Wa2'].reshape(1, 128)

    src = jnp.pad(edge_index[0].astype(jnp.int32), (0, E_PAD - E))
    dst_f = jnp.pad(edge_index[1].astype(jnp.int32), (0, E_PAD - E))
    ew = jnp.pad(edge_weight, (0, E_PAD - E))

    xT = jnp.pad(x, ((0, 0), (0, 76))).reshape(N, 14, 128) \
        .transpose(1, 0, 2).reshape(14 * N, 128)

    l_idx = jnp.pad(ddi_pairs[:, 0], (0, P_PAD - P))
    r_idx = jnp.pad(ddi_pairs[:, 1], (0, P_PAD - P))

    enc_h1, enc_h2, z = pl.pallas_call(
        _enc_body,
        grid=(GRID_M,),
        in_specs=[pl.BlockSpec((BM, 1716), lambda i: (i, 0))]
        + _mm_specs([(1716, 2048), (1, 2048), (2048, 256), (1, 256),
                     (256, 128), (1, 128)]),
        out_specs=[pl.BlockSpec((BM, 2048), lambda i: (i, 0)),
                   pl.BlockSpec((BM, 256), lambda i: (i, 0)),
                   pl.BlockSpec((BM, 128), lambda i: (i, 0))],
        out_shape=[jax.ShapeDtypeStruct((N, 2048), f32),
                   jax.ShapeDtypeStruct((N, 256), f32),
                   jax.ShapeDtypeStruct((N, 128), f32)],
    )(x, We1, be1, We2, be2, Wz, bz)

    x_bar = pl.pallas_call(
        _dec_body,
        grid=(GRID_M,),
        in_specs=[pl.BlockSpec((BM, 128), lambda i: (i, 0))]
        + _mm_specs([(128, 256), (1, 256), (256, 2048), (1, 2048),
                     (2048, 1716), (1, 1716)]),
        out_specs=pl.BlockSpec((BM, 1716), lambda i: (i, 0)),
        out_shape=jax.ShapeDtypeStruct((N, 1716), f32),
    )(z, Wd1, bd1, Wd2, bd2, Wxb, bxb)

    aggx = _agg(14)(src, dst_f, ew, xT).reshape(14, N, 128)

    sup2 = pl.pallas_call(
        _g1_body,
        grid=(GRID_M,),
        in_specs=[pl.BlockSpec((14, BM, 128), lambda i: (0, i, 0)),
                  pl.BlockSpec((BM, 2048), lambda i: (i, 0))]
        + _mm_specs([(14, 128, 2048), (2048, 256)]),
        out_specs=pl.BlockSpec((BM, 256), lambda i: (i, 0)),
        out_shape=jax.ShapeDtypeStruct((N, 256), f32),
    )(aggx, enc_h1, Wg1, Wg2)

    sup2T = sup2.reshape(N, 2, 128).transpose(1, 0, 2).reshape(2 * N, 128)
    agg2 = _agg(2)(src, dst_f, ew, sup2T).reshape(2, N, 128)

    sup3 = pl.pallas_call(
        _g3_body,
        grid=(GRID_M,),
        in_specs=[pl.BlockSpec((2, BM, 128), lambda i: (0, i, 0)),
                  pl.BlockSpec((BM, 256), lambda i: (i, 0))]
        + _mm_specs([(2, 128, 128)]),
        out_specs=pl.BlockSpec((BM, 128), lambda i: (i, 0)),
        out_shape=jax.ShapeDtypeStruct((N, 128), f32),
    )(agg2, enc_h2, Wg3)

    agg3 = _agg(1)(src, dst_f, ew, sup3).reshape(2, N, 128)

    emb1, beta2 = pl.pallas_call(
        _att_body,
        grid=(GRID_M,),
        in_specs=[pl.BlockSpec((2, BM, 128), lambda i: (0, i, 0)),
                  pl.BlockSpec((BM, 128), lambda i: (i, 0))]
        + _mm_specs([(128, 128), (1, 128), (1, 128)]),
        out_specs=[pl.BlockSpec((BM, 128), lambda i: (i, 0)),
                   pl.BlockSpec((BM, 2), lambda i: (i, 0))],
        out_shape=[jax.ShapeDtypeStruct((N, 128), f32),
                   jax.ShapeDtypeStruct((N, 2), f32)],
    )(agg3, z, Wa1, ba1, Wa2)

    beta = beta2.reshape(N, 2, 1)

    Bfull = _pair_mean(l_idx, r_idx, emb1)
    C1 = Bfull[:NUM_TRAIN]
    C2 = Bfull[NUM_TRAIN:P]

    return (emb1, beta, x_bar, C1, C2, labels[:NUM_TRAIN], labels[NUM_TRAIN:])
